# Initial kernel scaffold; baseline (speedup 1.0000x reference)
import jax
import jax.numpy as jnp
from jax.experimental import pallas as pl


def kernel(node_features, node_attrs, edge_embedding, edge_attrs, edge_index, W1, Wr1, Wr2, W2, Wsc):
    raise NotImplementedError("write your pallas kernel here")


# speedup vs baseline: 2.1999x; 2.1999x over previous
import functools

import jax
import jax.numpy as jnp
import numpy as np
from jax import lax
from jax.experimental import pallas as pl
from jax.experimental.pallas import tpu as pltpu
from jax.experimental.pallas import tpu_sc as plsc

_N = 10000
_E = 320000
_D = 128
_DA = 8
_DE = 16
_FCH = 8
_LOG2 = float(np.log(2.0))
_INV_SQRT_D = float(1.0 / np.sqrt(float(_D)))
_INV_SQRT_DE = float(1.0 / np.sqrt(float(_DE)))
_INV_SQRT_FCH = float(1.0 / np.sqrt(float(_FCH)))
_INV_SQRT_AVG = float(1.0 / np.sqrt(32.0))
_INV_SQRT_DDA = float(1.0 / np.sqrt(float(_D * _DA)))

_BN = 2000
_BE = 2000

_NC = 2
_NS = 16
_NW = _NC * _NS
_EPW = _E // _NW
_K = 80
_NCH = _EPW // _K
_NPAD = 10240
_RPT = _NPAD // _NS


def _ssp(v):
    return jnp.maximum(v, 0.0) + jnp.log1p(jnp.exp(-jnp.abs(v))) - _LOG2


def _x_body(nf_ref, w1_ref, x_ref):
    x_ref[...] = jnp.dot(nf_ref[...], w1_ref[...],
                         preferred_element_type=jnp.float32) * _INV_SQRT_D


def _x_call(nf, W1):
    return pl.pallas_call(
        _x_body,
        grid=(_N // _BN,),
        in_specs=[
            pl.BlockSpec((_BN, _D), lambda i: (i, 0)),
            pl.BlockSpec((_D, _D), lambda i: (0, 0)),
        ],
        out_specs=pl.BlockSpec((_BN, _D), lambda i: (i, 0)),
        out_shape=jax.ShapeDtypeStruct((_N, _D), jnp.float32),
    )(nf, W1)


def _w_body(ee_ref, a_ref, wr1_ref, wr2_ref, out_ref):
    h = jnp.dot(ee_ref[...], wr1_ref[...],
                preferred_element_type=jnp.float32) * _INV_SQRT_DE
    h = _ssp(h)
    w = jnp.dot(h, wr2_ref[...],
                preferred_element_type=jnp.float32) * (_INV_SQRT_FCH * _INV_SQRT_AVG)
    out_ref[...] = w * a_ref[...]


def _w_call(ee, ea, Wr1, Wr2):
    return pl.pallas_call(
        _w_body,
        grid=(_E // _BE,),
        in_specs=[
            pl.BlockSpec((_BE, _DE), lambda i: (i, 0)),
            pl.BlockSpec((_BE, 1), lambda i: (i, 0)),
            pl.BlockSpec((_DE, _FCH), lambda i: (0, 0)),
            pl.BlockSpec((_FCH, _D), lambda i: (0, 0)),
        ],
        out_specs=pl.BlockSpec((_BE, _D), lambda i: (i, 0)),
        out_shape=jax.ShapeDtypeStruct((_E, _D), jnp.float32),
    )(ee, ea, Wr1, Wr2)


def _sc_body(x_hbm, wp_hbm, src_hbm, dst_hbm, out_hbm,
             src_v, dst_v, rows_v, w_v, zbuf, agg_sh, sem_g):
    cid = lax.axis_index("c")
    sid = lax.axis_index("s")
    wid = sid * _NC + cid

    def _zb(k, carry):
        for j in range(_D // 16):
            zbuf[k, pl.ds(j * 16, 16)] = jnp.zeros((16,), jnp.float32)
        return carry
    lax.fori_loop(0, 128, _zb, 0)
    for c in range(_RPT // 128):
        pltpu.sync_copy(zbuf, agg_sh.at[pl.ds(sid * _RPT + c * 128, 128)])
    plsc.subcore_barrier()

    base = wid * _EPW

    def _chunk(t, carry):
        off = base + t * _K
        pltpu.sync_copy(src_hbm.at[pl.ds(off, _K)], src_v)
        gcp = pltpu.async_copy(x_hbm.at[src_v], rows_v, sem_g)
        pltpu.sync_copy(dst_hbm.at[pl.ds(off, _K)], dst_v)
        pltpu.sync_copy(wp_hbm.at[pl.ds(off, _K)], w_v)
        gcp.wait()

        def _mul(k, c2):
            for j in range(_D // 16):
                s = pl.ds(j * 16, 16)
                rows_v[k, s] = rows_v[k, s] * w_v[k, s]
            return c2
        lax.fori_loop(0, _K, _mul, 0)
        pltpu.sync_copy(rows_v, agg_sh.at[dst_v], add=True)
        return carry

    lax.fori_loop(0, _NCH, _chunk, 0)
    plsc.subcore_barrier()
    pltpu.sync_copy(agg_sh.at[pl.ds(sid * _RPT, _RPT)],
                    out_hbm.at[cid, pl.ds(sid * _RPT, _RPT)])


def _sc_call(x, wp, src, dst):
    mesh = plsc.VectorSubcoreMesh(core_axis_name="c", subcore_axis_name="s")
    f = pl.kernel(
        _sc_body,
        out_type=jax.ShapeDtypeStruct((_NC, _NPAD, _D), jnp.float32),
        mesh=mesh,
        scratch_types=[
            pltpu.VMEM((_K,), jnp.int32),
            pltpu.VMEM((_K,), jnp.int32),
            pltpu.VMEM((_K, _D), jnp.float32),
            pltpu.VMEM((_K, _D), jnp.float32),
            pltpu.VMEM((128, _D), jnp.float32),
            pltpu.VMEM_SHARED((_NPAD, _D), jnp.float32),
            pltpu.SemaphoreType.DMA,
        ],
    )
    return f(x, wp, src, dst)


def _final_body(nf_ref, na_ref, p_ref, w2_ref, wsct_ref, out_ref):
    agg = p_ref[0, :, :] + p_ref[1, :, :]
    out_lin = jnp.dot(agg, w2_ref[...],
                      preferred_element_type=jnp.float32) * _INV_SQRT_D
    nf = nf_ref[...]
    na = na_ref[...]
    sc = jnp.zeros_like(out_lin)
    for v in range(_DA):
        sc = sc + jnp.dot(nf * na[:, v:v + 1], wsct_ref[v],
                          preferred_element_type=jnp.float32)
    conv = out_lin + sc * _INV_SQRT_DDA
    out_ref[...] = nf + _ssp(conv)


def _final_call(nf, na, partials, W2, Wsc_t):
    return pl.pallas_call(
        _final_body,
        grid=(_N // _BN,),
        in_specs=[
            pl.BlockSpec((_BN, _D), lambda i: (i, 0)),
            pl.BlockSpec((_BN, _DA), lambda i: (i, 0)),
            pl.BlockSpec((_NC, _BN, _D), lambda i: (0, i, 0)),
            pl.BlockSpec((_D, _D), lambda i: (0, 0)),
            pl.BlockSpec((_DA, _D, _D), lambda i: (0, 0, 0)),
        ],
        out_specs=pl.BlockSpec((_BN, _D), lambda i: (i, 0)),
        out_shape=jax.ShapeDtypeStruct((_N, _D), jnp.float32),
    )(nf, na, partials, W2, Wsc_t)


def kernel(node_features, node_attrs, edge_embedding, edge_attrs, edge_index,
           W1, Wr1, Wr2, W2, Wsc):
    src = edge_index[0]
    dst = edge_index[1]
    x = _x_call(node_features, W1)
    wp = _w_call(edge_embedding, edge_attrs, Wr1, Wr2)
    partials = _sc_call(x, wp, src, dst)
    Wsc_t = jnp.transpose(Wsc, (1, 0, 2))
    return _final_call(node_features, node_attrs, partials, W2, Wsc_t)
